# baseline (speedup 1.0000x reference)
import jax
import jax.numpy as jnp
from jax.experimental import pallas as pl
from jax.experimental.pallas import tpu as pltpu

_N, _M, _D = 10000, 64, 32
_BN = 1000
_NB = _N // _BN

_F32 = jnp.float32
_CONTRACT0 = (((0,), (0,)), ((), ()))


def _fused_kernel(x0_ref, b_ref, w1_ref, w2_ref, wm1_ref, bm1_ref,
                  wm2_ref, bm2_ref, wm3_ref, bm3_ref, out_ref,
                  acc1_ref, accg_ref, t_ref, browc_ref):
    p = pl.program_id(0)
    j = pl.program_id(1)

    @pl.when(p == 0)
    def _reduce():
        B = b_ref[...]
        x0 = x0_ref[...]
        part1 = jax.lax.dot_general(B, x0, _CONTRACT0,
                                    preferred_element_type=_F32)
        partg = jax.lax.dot_general(B, B, _CONTRACT0,
                                    preferred_element_type=_F32)

        @pl.when(j == 0)
        def _init():
            acc1_ref[...] = part1
            accg_ref[...] = partg

        @pl.when(j > 0)
        def _acc():
            acc1_ref[...] += part1
            accg_ref[...] += partg

    @pl.when((p == 0) & (j == _NB - 1))
    def _finalize():
        x1_1 = acc1_ref[...]
        G = accg_ref[...]
        x1_2 = jnp.dot(G, jnp.dot(x1_1, w1_ref[...],
                                  preferred_element_type=_F32),
                       preferred_element_type=_F32)
        y = jnp.dot(x1_2, w2_ref[...], preferred_element_type=_F32)
        u = jnp.dot(wm2_ref[...], wm3_ref[...],
                    preferred_element_type=_F32)
        wm1 = wm1_ref[...]
        av = jnp.dot(wm1[:_D, :], u, preferred_element_type=_F32)
        bv = jnp.dot(wm1[_D:, :], u, preferred_element_type=_F32)
        c = (jnp.dot(bm1_ref[...][None, :], u,
                     preferred_element_type=_F32)[0, 0]
             + jnp.dot(bm2_ref[...][None, :], wm3_ref[...],
                       preferred_element_type=_F32)[0, 0]
             + bm3_ref[0])
        t_ref[...] = jnp.dot(y, av, preferred_element_type=_F32)
        browc_ref[...] = jax.lax.dot_general(
            bv, x1_2, (((0,), (1,)), ((), ())),
            preferred_element_type=_F32) + c

    @pl.when(p == 1)
    def _emit():
        B = b_ref[...]
        a_col = jnp.dot(B, t_ref[...], preferred_element_type=_F32)
        vals = a_col + browc_ref[...]
        out_ref[...] = jnp.where(B != 0, vals, 0.0)


def kernel(x_0, incidence_1, W1, W2, Wm1, bm1, Wm2, bm2, Wm3, bm3):
    n, m = incidence_1.shape
    d = x_0.shape[1]
    full = lambda a: pl.BlockSpec(a.shape, lambda p, j: (0,) * a.ndim)
    return pl.pallas_call(
        _fused_kernel,
        grid=(2, _NB),
        in_specs=[
            pl.BlockSpec((_BN, d), lambda p, j: (j * (1 - p), 0)),
            pl.BlockSpec((_BN, m), lambda p, j: (j, 0)),
            full(W1), full(W2), full(Wm1), full(bm1),
            full(Wm2), full(bm2), full(Wm3), full(bm3),
        ],
        out_specs=pl.BlockSpec((_BN, m), lambda p, j: (j * p, 0)),
        out_shape=jax.ShapeDtypeStruct((n, m), jnp.float32),
        scratch_shapes=[
            pltpu.VMEM((m, d), jnp.float32),
            pltpu.VMEM((m, m), jnp.float32),
            pltpu.VMEM((m, 1), jnp.float32),
            pltpu.VMEM((1, m), jnp.float32),
        ],
    )(x_0, incidence_1, W1, W2, Wm1, bm1, Wm2, bm2, Wm3, bm3)


# speedup vs baseline: 1.3240x; 1.3240x over previous
import jax
import jax.numpy as jnp
from jax.experimental import pallas as pl

_D = 32
_F32 = jnp.float32
_CONTRACT0 = (((0,), (0,)), ((), ()))


def _fused_kernel(x0_ref, b_ref, w1_ref, w2_ref, wm1_ref, bm1_ref,
                  wm2_ref, bm2_ref, wm3_ref, bm3_ref, out_ref):
    B = b_ref[...]
    x0 = x0_ref[...]

    x1_1 = jax.lax.dot_general(B, x0, _CONTRACT0,
                               preferred_element_type=_F32)
    G = jax.lax.dot_general(B, B, _CONTRACT0,
                            preferred_element_type=_F32)
    x1_2 = jnp.dot(G, jnp.dot(x1_1, w1_ref[...],
                              preferred_element_type=_F32),
                   preferred_element_type=_F32)
    y = jnp.dot(x1_2, w2_ref[...], preferred_element_type=_F32)

    u = jnp.dot(wm2_ref[...], wm3_ref[...],
                preferred_element_type=_F32)
    wm1 = wm1_ref[...]
    av = jnp.dot(wm1[:_D, :], u, preferred_element_type=_F32)
    bv = jnp.dot(wm1[_D:, :], u, preferred_element_type=_F32)
    c = (jnp.dot(bm1_ref[...][None, :], u,
                 preferred_element_type=_F32)[0, 0]
         + jnp.dot(bm2_ref[...][None, :], wm3_ref[...],
                   preferred_element_type=_F32)[0, 0]
         + bm3_ref[0])

    t = jnp.dot(y, av, preferred_element_type=_F32)
    browc = jax.lax.dot_general(bv, x1_2, (((0,), (1,)), ((), ())),
                                preferred_element_type=_F32) + c

    a_col = jnp.dot(B, t, preferred_element_type=_F32)
    out_ref[...] = jnp.where(B != 0, a_col + browc, 0.0)


def kernel(x_0, incidence_1, W1, W2, Wm1, bm1, Wm2, bm2, Wm3, bm3):
    n, m = incidence_1.shape
    return pl.pallas_call(
        _fused_kernel,
        out_shape=jax.ShapeDtypeStruct((n, m), jnp.float32),
    )(x_0, incidence_1, W1, W2, Wm1, bm1, Wm2, bm2, Wm3, bm3)
